# baseline (speedup 1.0000x reference)
import functools

import jax
import jax.numpy as jnp
from jax import lax
from jax.experimental import pallas as pl
from jax.experimental.pallas import tpu as pltpu
from jax.experimental.pallas import tpu_sc as plsc

R, C = 4096, 1024
BR = 512
NB = R // BR
N_PIX = R * C
K_COUNT = int(N_PIX * 10 / 100)
EPSILON = 3.1

NW = 32
PER_W = N_PIX // NW
CH = 4096
H1 = 65536
H2 = 32768

_sc_mesh = plsc.VectorSubcoreMesh(core_axis_name="c", subcore_axis_name="s")
_sc_params = pltpu.CompilerParams(needs_layout_passes=False)


def _k1_body(p_ref, g_ref, q_ref, sums_ref, acc_ref):
    b = pl.program_id(0)

    @pl.when(b == 0)
    def _():
        acc_ref[0] = 0.0
        acc_ref[1] = 0.0
        acc_ref[2] = 0.0

    p = p_ref[...]
    g = g_ref[...]
    q_ref[...] = jnp.where(g == 1.0, p, 1.0 - p)
    acc_ref[0] += jnp.sum(p * g)
    acc_ref[1] += jnp.sum(p)
    acc_ref[2] += jnp.sum(g)

    @pl.when(b == NB - 1)
    def _():
        row = lax.broadcasted_iota(jnp.int32, (8, 128), 0)
        lane = lax.broadcasted_iota(jnp.int32, (8, 128), 1)
        v = jnp.where((row == 0) & (lane == 0), acc_ref[0], 0.0)
        v = jnp.where((row == 0) & (lane == 1), acc_ref[1], v)
        v = jnp.where((row == 0) & (lane == 2), acc_ref[2], v)
        sums_ref[...] = v


def _sc_hist1(q_hbm, hist_hbm, h_v):
    cid = lax.axis_index("c")
    sid = lax.axis_index("s")
    wid = sid * 2 + cid

    @pl.loop(0, H1, step=128)
    def _(i):
        for j in range(8):
            h_v[pl.ds(i + j * 16, 16)] = jnp.zeros((16,), jnp.int32)

    ones = jnp.full((16,), 1, jnp.int32)

    def body(q_v):
        @pl.loop(0, CH, step=128)
        def _(i):
            for j in range(8):
                v = q_v[pl.ds(i + j * 16, 16)]
                bits = plsc.bitcast(v, jnp.int32)
                idx = lax.shift_right_logical(bits, 15)
                plsc.addupdate_scatter(h_v, [idx], ones)

    pltpu.emit_pipeline(
        body,
        grid=(N_PIX // CH,),
        in_specs=[pl.BlockSpec((CH,), lambda i: (i,))],
        core_axis_name=("c", "s"),
        dimension_semantics=(pltpu.PARALLEL,),
    )(q_hbm)
    pltpu.sync_copy(h_v, hist_hbm.at[wid])


def _sc_hist2(q_hbm, b1_hbm, hist_hbm, h_v, b_v):
    cid = lax.axis_index("c")
    sid = lax.axis_index("s")
    wid = sid * 2 + cid

    @pl.loop(0, H2, step=128)
    def _(i):
        for j in range(8):
            h_v[pl.ds(i + j * 16, 16)] = jnp.zeros((16,), jnp.int32)

    pltpu.sync_copy(b1_hbm.at[pl.ds(0, 16)], b_v)
    hb = b_v[...]
    ones = jnp.full((16,), 1, jnp.int32)
    low_mask = jnp.full((16,), H2 - 1, jnp.int32)

    def body(q_v):
        @pl.loop(0, CH, step=128)
        def _(i):
            for j in range(8):
                v = q_v[pl.ds(i + j * 16, 16)]
                bits = plsc.bitcast(v, jnp.int32)
                hi = lax.shift_right_logical(bits, 15)
                low = bits & low_mask
                plsc.addupdate_scatter(h_v, [low], ones, mask=hi == hb)

    pltpu.emit_pipeline(
        body,
        grid=(N_PIX // CH,),
        in_specs=[pl.BlockSpec((CH,), lambda i: (i,))],
        core_axis_name=("c", "s"),
        dimension_semantics=(pltpu.PARALLEL,),
    )(q_hbm)
    pltpu.sync_copy(h_v, hist_hbm.at[wid])


def _make_scan_body(rows, nbuckets, with_prev):
    def scan_body(*refs):
        if with_prev:
            prev_ref, h_ref, out_ref, hacc_ref = refs
        else:
            h_ref, out_ref, hacc_ref = refs
        b = pl.program_id(0)

        @pl.when(b == 0)
        def _():
            hacc_ref[...] = jnp.zeros((rows, 128), jnp.int32)

        hacc_ref[...] += h_ref[0]

        @pl.when(b == NW - 1)
        def _():
            h = hacc_ref[...]
            flat = (lax.broadcasted_iota(jnp.int32, (rows, 128), 0) * 128
                    + lax.broadcasted_iota(jnp.int32, (rows, 128), 1))
            if with_prev:
                pv = prev_ref[...]
                row = lax.broadcasted_iota(jnp.int32, (8, 128), 0)
                prev_b = jnp.sum(jnp.where(row == 0, pv, 0)) // 128
                prev_below = jnp.sum(jnp.where(row == 1, pv, 0)) // 128
                target = K_COUNT - prev_below
            else:
                prev_b = jnp.int32(0)
                prev_below = jnp.int32(0)
                target = jnp.int32(K_COUNT)

            def step(_, lohi):
                lo, hi = lohi
                mid = (lo + hi) // 2
                s = jnp.sum(jnp.where(flat <= mid, h, 0))
                return jnp.where(s >= target, lo, mid + 1), \
                       jnp.where(s >= target, mid, hi)

            nbits = nbuckets.bit_length() - 1
            lo, hi = lax.fori_loop(0, nbits + 1, step,
                                   (jnp.int32(0), jnp.int32(nbuckets - 1)))
            bkt = lo
            below = jnp.sum(jnp.where(flat < bkt, h, 0))
            out_b = prev_b * nbuckets + bkt
            out_below = prev_below + below
            row = lax.broadcasted_iota(jnp.int32, (8, 128), 0)
            out = jnp.where(row == 0, out_b, 0)
            out = jnp.where(row == 1, out_below, out)
            out_ref[...] = out

    return scan_body


def _scan_call(hist, rows, nbuckets, prev=None):
    h3 = hist.reshape(NW, rows, 128)
    in_specs = [pl.BlockSpec((1, rows, 128), lambda b: (b, 0, 0))]
    args = [h3]
    if prev is not None:
        in_specs = [pl.BlockSpec((8, 128), lambda b: (0, 0))] + in_specs
        args = [prev] + args
    return pl.pallas_call(
        _make_scan_body(rows, nbuckets, prev is not None),
        grid=(NW,),
        in_specs=in_specs,
        out_specs=pl.BlockSpec((8, 128), lambda b: (0, 0)),
        out_shape=jax.ShapeDtypeStruct((8, 128), jnp.int32),
        scratch_shapes=[pltpu.VMEM((rows, 128), jnp.int32)],
        compiler_params=pltpu.CompilerParams(
            dimension_semantics=("arbitrary",)),
    )(*args)


def _k5_body(sums_ref, tn_ref, q_ref, out_ref, fa_ref):
    b = pl.program_id(0)

    @pl.when(b == 0)
    def _():
        fa_ref[0] = 0.0

    row = lax.broadcasted_iota(jnp.int32, (8, 128), 0)
    tn = tn_ref[...]
    tbits = jnp.sum(jnp.where(row == 0, tn, 0)) // 128
    n_less = jnp.sum(jnp.where(row == 1, tn, 0)) // 128

    q = q_ref[...]
    bits = lax.bitcast_convert_type(q, jnp.int32)
    bce = -jnp.maximum(jnp.log(q), -100.0)
    poly = bce + (1.0 - jnp.exp(-bce)) * EPSILON
    fa_ref[0] += jnp.sum(jnp.where(bits < tbits, poly, 0.0))

    @pl.when(b == NB - 1)
    def _():
        lane = lax.broadcasted_iota(jnp.int32, (8, 128), 1)
        s = sums_ref[...]
        inter = jnp.sum(jnp.where((row == 0) & (lane == 0), s, 0.0))
        sum_p = jnp.sum(jnp.where((row == 0) & (lane == 1), s, 0.0))
        sum_g = jnp.sum(jnp.where((row == 0) & (lane == 2), s, 0.0))
        dice = 1.0 - (2.0 * inter + 1.0) / (sum_p + sum_g + 1.0)

        tq = lax.bitcast_convert_type(
            jnp.full((8, 128), tbits, jnp.int32), jnp.float32)
        bce_t = -jnp.maximum(jnp.log(tq), -100.0)
        poly_t = bce_t + (1.0 - jnp.exp(-bce_t)) * EPSILON
        total = fa_ref[0] + (K_COUNT - n_less).astype(jnp.float32) * poly_t
        out_ref[...] = dice + total / jnp.float32(K_COUNT)


def kernel(preds, gt_masks):
    p2 = preds.reshape(R, C)
    g2 = gt_masks.reshape(R, C)

    q, sums = pl.pallas_call(
        _k1_body,
        grid=(NB,),
        in_specs=[
            pl.BlockSpec((BR, C), lambda b: (b, 0)),
            pl.BlockSpec((BR, C), lambda b: (b, 0)),
        ],
        out_specs=[
            pl.BlockSpec((BR, C), lambda b: (b, 0)),
            pl.BlockSpec((8, 128), lambda b: (0, 0)),
        ],
        out_shape=[
            jax.ShapeDtypeStruct((R, C), jnp.float32),
            jax.ShapeDtypeStruct((8, 128), jnp.float32),
        ],
        scratch_shapes=[pltpu.SMEM((4,), jnp.float32)],
        compiler_params=pltpu.CompilerParams(
            dimension_semantics=("arbitrary",)),
    )(p2, g2)

    qf = q.reshape(N_PIX)

    hist1 = functools.partial(
        pl.kernel,
        out_type=jax.ShapeDtypeStruct((NW, H1), jnp.int32),
        mesh=_sc_mesh,
        scratch_types=[pltpu.VMEM((H1,), jnp.int32)],
        compiler_params=_sc_params,
    )(_sc_hist1)(qf)

    b1n = _scan_call(hist1, H1 // 128, H1)
    b1f = b1n.reshape(8 * 128)

    hist2 = functools.partial(
        pl.kernel,
        out_type=jax.ShapeDtypeStruct((NW, H2), jnp.int32),
        mesh=_sc_mesh,
        scratch_types=[
            pltpu.VMEM((H2,), jnp.int32),
            pltpu.VMEM((16,), jnp.int32),
        ],
        compiler_params=_sc_params,
    )(_sc_hist2)(qf, b1f)

    tn = _scan_call(hist2, H2 // 128, H2, prev=b1n)

    out = pl.pallas_call(
        _k5_body,
        grid=(NB,),
        in_specs=[
            pl.BlockSpec((8, 128), lambda b: (0, 0)),
            pl.BlockSpec((8, 128), lambda b: (0, 0)),
            pl.BlockSpec((BR, C), lambda b: (b, 0)),
        ],
        out_specs=pl.BlockSpec((8, 128), lambda b: (0, 0)),
        out_shape=jax.ShapeDtypeStruct((8, 128), jnp.float32),
        scratch_shapes=[pltpu.SMEM((2,), jnp.float32)],
        compiler_params=pltpu.CompilerParams(
            dimension_semantics=("arbitrary",)),
    )(sums, tn, q)

    return out[0, 0]


# speedup vs baseline: 1.3475x; 1.3475x over previous
import functools

import jax
import jax.numpy as jnp
from jax import lax
from jax.experimental import pallas as pl
from jax.experimental.pallas import tpu as pltpu
from jax.experimental.pallas import tpu_sc as plsc

N_PIX = 16 * 512 * 512
BLK = 512 * 1024
NB = N_PIX // BLK
K_COUNT = int(N_PIX * 10 / 100)
EPSILON = 3.1

NW = 32
CH = 4096
H1 = 65536
H2 = 32768

_sc_mesh = plsc.VectorSubcoreMesh(core_axis_name="c", subcore_axis_name="s")
_sc_params = pltpu.CompilerParams(needs_layout_passes=False)


def _k1_body(p_ref, g_ref, q_ref, sums_ref, acc_ref):
    b = pl.program_id(0)

    @pl.when(b == 0)
    def _():
        acc_ref[0] = 0.0
        acc_ref[1] = 0.0
        acc_ref[2] = 0.0

    p = p_ref[...]
    g = g_ref[...]
    q_ref[...] = jnp.where(g == 1.0, p, 1.0 - p)
    acc_ref[0] += jnp.sum(p * g)
    acc_ref[1] += jnp.sum(p)
    acc_ref[2] += jnp.sum(g)

    @pl.when(b == NB - 1)
    def _():
        row = lax.broadcasted_iota(jnp.int32, (8, 128), 0)
        lane = lax.broadcasted_iota(jnp.int32, (8, 128), 1)
        v = jnp.where((row == 0) & (lane == 0), acc_ref[0], 0.0)
        v = jnp.where((row == 0) & (lane == 1), acc_ref[1], v)
        v = jnp.where((row == 0) & (lane == 2), acc_ref[2], v)
        sums_ref[...] = v


def _sc_hist1(q_hbm, hist_hbm, h_v):
    cid = lax.axis_index("c")
    sid = lax.axis_index("s")
    wid = sid * 2 + cid

    zeros = jnp.zeros((16,), jnp.int32)

    @plsc.parallel_loop(0, H1 // 128, unroll=4)
    def _(i):
        for j in range(8):
            h_v[i, pl.ds(j * 16, 16)] = zeros

    ones = jnp.full((16,), 1, jnp.int32)
    lmask = jnp.full((16,), 127, jnp.int32)

    def body(q_v):
        @plsc.parallel_loop(0, CH, step=16, unroll=8)
        def _(i):
            v = q_v[pl.ds(i, 16)]
            bits = plsc.bitcast(v, jnp.int32)
            idx = lax.shift_right_logical(bits, 15)
            plsc.addupdate_scatter(
                h_v, [lax.shift_right_logical(idx, 7), idx & lmask], ones)

    pltpu.emit_pipeline(
        body,
        grid=(N_PIX // CH,),
        in_specs=[pl.BlockSpec((CH,), lambda i: (i,))],
        core_axis_name=("c", "s"),
        dimension_semantics=(pltpu.PARALLEL,),
    )(q_hbm)
    pltpu.sync_copy(h_v, hist_hbm.at[wid])


def _sc_hist2(q_hbm, b1_hbm, hist_hbm, h_v, b_v):
    cid = lax.axis_index("c")
    sid = lax.axis_index("s")
    wid = sid * 2 + cid

    zeros = jnp.zeros((16,), jnp.int32)

    @plsc.parallel_loop(0, H2 // 128, unroll=4)
    def _(i):
        for j in range(8):
            h_v[i, pl.ds(j * 16, 16)] = zeros

    pltpu.sync_copy(b1_hbm.at[pl.ds(0, 16)], b_v)
    hb = b_v[...]
    ones = jnp.full((16,), 1, jnp.int32)
    lmask = jnp.full((16,), 127, jnp.int32)

    def body(q_v):
        @plsc.parallel_loop(0, CH, step=16, unroll=8)
        def _(i):
            v = q_v[pl.ds(i, 16)]
            bits = plsc.bitcast(v, jnp.int32)
            hi = lax.shift_right_logical(bits, 15)
            low = lax.shift_right_logical(bits, 7) & jnp.full((16,), 255,
                                                             jnp.int32)
            plsc.addupdate_scatter(
                h_v, [low, bits & lmask], ones, mask=hi == hb)

    pltpu.emit_pipeline(
        body,
        grid=(N_PIX // CH,),
        in_specs=[pl.BlockSpec((CH,), lambda i: (i,))],
        core_axis_name=("c", "s"),
        dimension_semantics=(pltpu.PARALLEL,),
    )(q_hbm)
    pltpu.sync_copy(h_v, hist_hbm.at[wid])


def _make_scan_body(rows, nbuckets, with_prev):
    def scan_body(*refs):
        if with_prev:
            prev_ref, h_ref, out_ref, hacc_ref = refs
        else:
            h_ref, out_ref, hacc_ref = refs
        b = pl.program_id(0)

        @pl.when(b == 0)
        def _():
            hacc_ref[...] = jnp.zeros((rows, 128), jnp.int32)

        hacc_ref[...] += h_ref[0]

        @pl.when(b == NW - 1)
        def _():
            h = hacc_ref[...]
            flat = (lax.broadcasted_iota(jnp.int32, (rows, 128), 0) * 128
                    + lax.broadcasted_iota(jnp.int32, (rows, 128), 1))
            if with_prev:
                pv = prev_ref[...]
                row = lax.broadcasted_iota(jnp.int32, (8, 128), 0)
                prev_b = jnp.sum(jnp.where(row == 0, pv, 0)) // 128
                prev_below = jnp.sum(jnp.where(row == 1, pv, 0)) // 128
                target = K_COUNT - prev_below
            else:
                prev_b = jnp.int32(0)
                prev_below = jnp.int32(0)
                target = jnp.int32(K_COUNT)

            def step(_, lohi):
                lo, hi = lohi
                mid = (lo + hi) // 2
                s = jnp.sum(jnp.where(flat <= mid, h, 0))
                return (jnp.where(s >= target, lo, mid + 1),
                        jnp.where(s >= target, mid, hi))

            nbits = nbuckets.bit_length() - 1
            lo, hi = lax.fori_loop(0, nbits + 1, step,
                                   (jnp.int32(0), jnp.int32(nbuckets - 1)))
            bkt = lo
            below = jnp.sum(jnp.where(flat < bkt, h, 0))
            out_b = prev_b * nbuckets + bkt
            out_below = prev_below + below
            row = lax.broadcasted_iota(jnp.int32, (8, 128), 0)
            out = jnp.where(row == 0, out_b, 0)
            out = jnp.where(row == 1, out_below, out)
            out_ref[...] = out

    return scan_body


def _scan_call(hist, rows, nbuckets, prev=None):
    in_specs = [pl.BlockSpec((1, rows, 128), lambda b: (b, 0, 0))]
    args = [hist]
    if prev is not None:
        in_specs = [pl.BlockSpec((8, 128), lambda b: (0, 0))] + in_specs
        args = [prev] + args
    return pl.pallas_call(
        _make_scan_body(rows, nbuckets, prev is not None),
        grid=(NW,),
        in_specs=in_specs,
        out_specs=pl.BlockSpec((8, 128), lambda b: (0, 0)),
        out_shape=jax.ShapeDtypeStruct((8, 128), jnp.int32),
        scratch_shapes=[pltpu.VMEM((rows, 128), jnp.int32)],
        compiler_params=pltpu.CompilerParams(
            dimension_semantics=("arbitrary",)),
    )(*args)


def _k5_body(sums_ref, tn_ref, q_ref, out_ref, fa_ref):
    b = pl.program_id(0)

    @pl.when(b == 0)
    def _():
        fa_ref[0] = 0.0

    row = lax.broadcasted_iota(jnp.int32, (8, 128), 0)
    tn = tn_ref[...]
    tbits = jnp.sum(jnp.where(row == 0, tn, 0)) // 128
    n_less = jnp.sum(jnp.where(row == 1, tn, 0)) // 128

    q = q_ref[...]
    bits = lax.bitcast_convert_type(q, jnp.int32)
    bce = -jnp.maximum(jnp.log(q), -100.0)
    poly = bce + (1.0 - jnp.exp(-bce)) * EPSILON
    fa_ref[0] += jnp.sum(jnp.where(bits < tbits, poly, 0.0))

    @pl.when(b == NB - 1)
    def _():
        lane = lax.broadcasted_iota(jnp.int32, (8, 128), 1)
        s = sums_ref[...]
        inter = jnp.sum(jnp.where((row == 0) & (lane == 0), s, 0.0))
        sum_p = jnp.sum(jnp.where((row == 0) & (lane == 1), s, 0.0))
        sum_g = jnp.sum(jnp.where((row == 0) & (lane == 2), s, 0.0))
        dice = 1.0 - (2.0 * inter + 1.0) / (sum_p + sum_g + 1.0)

        tq = lax.bitcast_convert_type(
            jnp.full((8, 128), tbits, jnp.int32), jnp.float32)
        bce_t = -jnp.maximum(jnp.log(tq), -100.0)
        poly_t = bce_t + (1.0 - jnp.exp(-bce_t)) * EPSILON
        total = fa_ref[0] + (K_COUNT - n_less).astype(jnp.float32) * poly_t
        out_ref[...] = dice + total / jnp.float32(K_COUNT)


def kernel(preds, gt_masks):
    pf = preds.reshape(N_PIX)
    gf = gt_masks.reshape(N_PIX)

    q, sums = pl.pallas_call(
        _k1_body,
        grid=(NB,),
        in_specs=[
            pl.BlockSpec((BLK,), lambda b: (b,)),
            pl.BlockSpec((BLK,), lambda b: (b,)),
        ],
        out_specs=[
            pl.BlockSpec((BLK,), lambda b: (b,)),
            pl.BlockSpec((8, 128), lambda b: (0, 0)),
        ],
        out_shape=[
            jax.ShapeDtypeStruct((N_PIX,), jnp.float32),
            jax.ShapeDtypeStruct((8, 128), jnp.float32),
        ],
        scratch_shapes=[pltpu.SMEM((4,), jnp.float32)],
        compiler_params=pltpu.CompilerParams(
            dimension_semantics=("arbitrary",)),
    )(pf, gf)

    hist1 = functools.partial(
        pl.kernel,
        out_type=jax.ShapeDtypeStruct((NW, H1 // 128, 128), jnp.int32),
        mesh=_sc_mesh,
        scratch_types=[pltpu.VMEM((H1 // 128, 128), jnp.int32)],
        compiler_params=_sc_params,
    )(_sc_hist1)(q)

    b1n = _scan_call(hist1, H1 // 128, H1)
    b1f = b1n.reshape(8 * 128)

    hist2 = functools.partial(
        pl.kernel,
        out_type=jax.ShapeDtypeStruct((NW, H2 // 128, 128), jnp.int32),
        mesh=_sc_mesh,
        scratch_types=[
            pltpu.VMEM((H2 // 128, 128), jnp.int32),
            pltpu.VMEM((16,), jnp.int32),
        ],
        compiler_params=_sc_params,
    )(_sc_hist2)(q, b1f)

    tn = _scan_call(hist2, H2 // 128, H2, prev=b1n)

    out = pl.pallas_call(
        _k5_body,
        grid=(NB,),
        in_specs=[
            pl.BlockSpec((8, 128), lambda b: (0, 0)),
            pl.BlockSpec((8, 128), lambda b: (0, 0)),
            pl.BlockSpec((BLK,), lambda b: (b,)),
        ],
        out_specs=pl.BlockSpec((8, 128), lambda b: (0, 0)),
        out_shape=jax.ShapeDtypeStruct((8, 128), jnp.float32),
        scratch_shapes=[pltpu.SMEM((2,), jnp.float32)],
        compiler_params=pltpu.CompilerParams(
            dimension_semantics=("arbitrary",)),
    )(sums, tn, q)

    return out[0, 0]
